# baseline (speedup 1.0000x reference)
import functools

import jax
import jax.numpy as jnp
import numpy as np
from jax import lax
from jax.experimental import pallas as pl
from jax.experimental.pallas import tpu as pltpu
from jax.experimental.pallas import tpu_sc as plsc

_DELTA = 5.0
_R_FRAC = 0.01
_INT_MIN = np.array(0x80000000, dtype=np.uint32).view(np.int32).item()
_BIT = [np.array(1 << b, dtype=np.uint32).view(np.int32).item()
        for b in range(31, -1, -1)]


def _softplus(x):
    return jnp.maximum(x, 0.0) + jnp.log1p(jnp.exp(-jnp.abs(x)))


def _tc_body(tgt_ref, x_ref, lneg_ref, possp_ref, *, k):
    x = x_ref[...]
    t = tgt_ref[...]
    rows = x.shape[0]
    col = lax.broadcasted_iota(jnp.int32, x.shape, 1)
    is_t = col == t

    pos_logit = jnp.sum(jnp.where(is_t, x, 0.0), axis=1)

    s = lax.bitcast_convert_type(x, jnp.int32)
    key = jnp.where(s >= 0, s, ~(s & jnp.int32(0x7FFFFFFF)))
    key = jnp.where(is_t, jnp.int32(_INT_MIN), key)

    kf = jnp.float32(k)
    ki = jnp.int32(k)
    pu = jnp.zeros((rows, 1), jnp.int32)
    for b in _BIT:
        cand = pu | jnp.int32(b)
        thr = cand ^ jnp.int32(_INT_MIN)
        c = jnp.sum(key >= thr, axis=1, keepdims=True, dtype=jnp.int32)
        pu = jnp.where(c >= ki, cand, pu)
    tau_key = pu ^ jnp.int32(_INT_MIN)

    sel = key > tau_key
    cnt_gt = jnp.sum(sel, axis=1, dtype=jnp.int32).astype(jnp.float32)
    sum_gt = jnp.sum(jnp.where(sel, _softplus(x), 0.0), axis=1)

    tau_bits = jnp.where(tau_key >= 0, tau_key,
                         (~tau_key) | jnp.int32(_INT_MIN))
    tau_f = lax.bitcast_convert_type(tau_bits, jnp.float32)[:, 0]

    l_neg = (sum_gt + (kf - cnt_gt) * _softplus(tau_f)) / kf
    lneg_ref[...] = jnp.broadcast_to(l_neg[:, None], (rows, 128))
    possp_ref[...] = jnp.broadcast_to(_softplus(-pos_logit)[:, None],
                                      (rows, 128))


def _neg_and_pos(logits, targets, k, rows_per_block=32):
    b, n = logits.shape
    grid = b // rows_per_block
    tgt2 = targets.reshape(b, 1)
    lneg, possp = pl.pallas_call(
        functools.partial(_tc_body, k=k),
        grid=(grid,),
        in_specs=[
            pl.BlockSpec((rows_per_block, 1), lambda i: (i, 0)),
            pl.BlockSpec((rows_per_block, n), lambda i: (i, 0)),
        ],
        out_specs=[
            pl.BlockSpec((rows_per_block, 128), lambda i: (i, 0)),
            pl.BlockSpec((rows_per_block, 128), lambda i: (i, 0)),
        ],
        out_shape=[
            jax.ShapeDtypeStruct((b, 128), jnp.float32),
            jax.ShapeDtypeStruct((b, 128), jnp.float32),
        ],
    )(tgt2, logits)
    return lneg[:, 0], possp[:, 0]


def _sc_gather_pos_cnt(co_cnts, targets):
    b, n = co_cnts.shape
    cf = co_cnts.reshape(-1)
    info = plsc.get_sparse_core_info()
    nw = info.num_cores * info.num_subcores
    bpw = b // nw
    mesh = plsc.VectorSubcoreMesh(core_axis_name="c", subcore_axis_name="s")

    @functools.partial(
        pl.kernel, mesh=mesh,
        out_type=jax.ShapeDtypeStruct((b,), jnp.float32),
        scratch_types=[
            pltpu.VMEM((bpw,), jnp.int32),
            pltpu.VMEM((bpw,), jnp.int32),
            pltpu.VMEM((bpw,), jnp.float32),
            pltpu.SemaphoreType.DMA,
        ],
    )
    def sck(cf_hbm, tgt_hbm, out_hbm, tgt_v, idx_v, val_v, sem):
        wid = lax.axis_index("s") * info.num_cores + lax.axis_index("c")
        base = wid * bpw
        pltpu.sync_copy(tgt_hbm.at[pl.ds(base, bpw)], tgt_v)
        for j in range(bpw // 16):
            tv = tgt_v[pl.ds(j * 16, 16)]
            row = (base + j * 16) + lax.iota(jnp.int32, 16)
            idx_v[pl.ds(j * 16, 16)] = tv + row * n
        pltpu.async_copy(cf_hbm.at[idx_v], val_v, sem).wait()
        pltpu.sync_copy(val_v, out_hbm.at[pl.ds(base, bpw)])

    return sck(cf, targets)


def kernel(logits, targets, co_cnts):
    b, n = logits.shape
    k = int(_R_FRAC * (n - 1))
    lneg, possp = _neg_and_pos(logits, targets, k)
    pos_cnt = _sc_gather_pos_cnt(co_cnts, targets)
    return jnp.mean(_DELTA * pos_cnt * possp + lneg)


# speedup vs baseline: 1.0087x; 1.0087x over previous
import functools

import jax
import jax.numpy as jnp
import numpy as np
from jax import lax
from jax.experimental import pallas as pl
from jax.experimental.pallas import tpu as pltpu
from jax.experimental.pallas import tpu_sc as plsc

_DELTA = 5.0
_R_FRAC = 0.01
_INT_MIN = np.array(0x80000000, dtype=np.uint32).view(np.int32).item()
_BIT = [np.array(1 << b, dtype=np.uint32).view(np.int32).item()
        for b in range(31, -1, -1)]


def _softplus(x):
    return jnp.maximum(x, 0.0) + jnp.log1p(jnp.exp(-jnp.abs(x)))


def _tc_body(tgt_ref, x_ref, lneg_ref, possp_ref, *, k):
    x = x_ref[...]
    t = tgt_ref[...]
    rows = x.shape[0]
    col = lax.broadcasted_iota(jnp.int32, x.shape, 1)
    is_t = col == t

    pos_logit = jnp.sum(jnp.where(is_t, x, 0.0), axis=1)

    s = lax.bitcast_convert_type(x, jnp.int32)
    key = jnp.where(s >= 0, s, ~(s & jnp.int32(0x7FFFFFFF)))
    key = jnp.where(is_t, jnp.int32(_INT_MIN), key)

    kf = jnp.float32(k)
    ki = jnp.int32(k)
    pu = jnp.zeros((rows, 1), jnp.int32)
    for b in _BIT:
        cand = pu | jnp.int32(b)
        thr = cand ^ jnp.int32(_INT_MIN)
        c = jnp.sum((key >= thr).astype(jnp.float32), axis=1, keepdims=True)
        pu = jnp.where(c >= kf, cand, pu)
    tau_key = pu ^ jnp.int32(_INT_MIN)

    sel = key > tau_key
    cnt_gt = jnp.sum(sel.astype(jnp.float32), axis=1)
    sum_gt = jnp.sum(jnp.where(sel, _softplus(x), 0.0), axis=1)

    tau_bits = jnp.where(tau_key >= 0, tau_key,
                         (~tau_key) | jnp.int32(_INT_MIN))
    tau_f = lax.bitcast_convert_type(tau_bits, jnp.float32)[:, 0]

    l_neg = (sum_gt + (kf - cnt_gt) * _softplus(tau_f)) / kf
    lneg_ref[...] = jnp.broadcast_to(l_neg[:, None], (rows, 128))
    possp_ref[...] = jnp.broadcast_to(_softplus(-pos_logit)[:, None],
                                      (rows, 128))


def _neg_and_pos(logits, targets, k, rows_per_block=32):
    b, n = logits.shape
    grid = b // rows_per_block
    tgt2 = targets.reshape(b, 1)
    lneg, possp = pl.pallas_call(
        functools.partial(_tc_body, k=k),
        grid=(grid,),
        in_specs=[
            pl.BlockSpec((rows_per_block, 1), lambda i: (i, 0)),
            pl.BlockSpec((rows_per_block, n), lambda i: (i, 0)),
        ],
        out_specs=[
            pl.BlockSpec((rows_per_block, 128), lambda i: (i, 0)),
            pl.BlockSpec((rows_per_block, 128), lambda i: (i, 0)),
        ],
        out_shape=[
            jax.ShapeDtypeStruct((b, 128), jnp.float32),
            jax.ShapeDtypeStruct((b, 128), jnp.float32),
        ],
    )(tgt2, logits)
    return lneg[:, 0], possp[:, 0]


def _sc_gather_pos_cnt(co_cnts, targets):
    b, n = co_cnts.shape
    cf = co_cnts.reshape(-1)
    info = plsc.get_sparse_core_info()
    nw = info.num_cores * info.num_subcores
    bpw = b // nw
    mesh = plsc.VectorSubcoreMesh(core_axis_name="c", subcore_axis_name="s")

    @functools.partial(
        pl.kernel, mesh=mesh,
        out_type=jax.ShapeDtypeStruct((b,), jnp.float32),
        scratch_types=[
            pltpu.VMEM((bpw,), jnp.int32),
            pltpu.VMEM((bpw,), jnp.int32),
            pltpu.VMEM((bpw,), jnp.float32),
            pltpu.SemaphoreType.DMA,
        ],
    )
    def sck(cf_hbm, tgt_hbm, out_hbm, tgt_v, idx_v, val_v, sem):
        wid = lax.axis_index("s") * info.num_cores + lax.axis_index("c")
        base = wid * bpw
        pltpu.sync_copy(tgt_hbm.at[pl.ds(base, bpw)], tgt_v)
        for j in range(bpw // 16):
            tv = tgt_v[pl.ds(j * 16, 16)]
            row = (base + j * 16) + lax.iota(jnp.int32, 16)
            idx_v[pl.ds(j * 16, 16)] = tv + row * n
        pltpu.async_copy(cf_hbm.at[idx_v], val_v, sem).wait()
        pltpu.sync_copy(val_v, out_hbm.at[pl.ds(base, bpw)])

    return sck(cf, targets)


def kernel(logits, targets, co_cnts):
    b, n = logits.shape
    k = int(_R_FRAC * (n - 1))
    lneg, possp = _neg_and_pos(logits, targets, k)
    pos_cnt = _sc_gather_pos_cnt(co_cnts, targets)
    return jnp.mean(_DELTA * pos_cnt * possp + lneg)


# speedup vs baseline: 1.1551x; 1.1451x over previous
import functools

import jax
import jax.numpy as jnp
import numpy as np
from jax import lax
from jax.experimental import pallas as pl
from jax.experimental.pallas import tpu as pltpu

_DELTA = 5.0
_R_FRAC = 0.01
_INT_MIN = np.array(0x80000000, dtype=np.uint32).view(np.int32).item()
_BIT = [np.array(1 << b, dtype=np.uint32).view(np.int32).item()
        for b in range(31, -1, -1)]
_WIN = 128


def _softplus(x):
    return jnp.maximum(x, 0.0) + jnp.log1p(jnp.exp(-jnp.abs(x)))


def _body(tgt_smem, tgt_ref, x_ref, cc_any, out_ref, win_ref, sem, *, k, n):
    rows = x_ref.shape[0]
    row0 = pl.program_id(0) * rows

    copies = []
    for r in range(rows):
        t = tgt_smem[r, 0]
        t0 = pl.multiple_of(lax.bitwise_and(t, jnp.int32(~127)), 128)
        cp = pltpu.make_async_copy(
            cc_any.at[pl.ds(pl.multiple_of(row0 + (r & ~7), 8), 8),
                      pl.ds(t0, _WIN)],
            win_ref.at[r],
            sem,
        )
        cp.start()
        copies.append(cp)

    x = x_ref[...]
    t = tgt_ref[...]
    col = lax.broadcasted_iota(jnp.int32, x.shape, 1)
    is_t = col == t

    pos_logit = jnp.sum(jnp.where(is_t, x, 0.0), axis=1)

    s = lax.bitcast_convert_type(x, jnp.int32)
    key = jnp.where(s >= 0, s, ~(s & jnp.int32(0x7FFFFFFF)))
    key = jnp.where(is_t, jnp.int32(_INT_MIN), key)

    kf = jnp.float32(k)
    pu = jnp.zeros((rows, 1), jnp.int32)
    for b in _BIT:
        cand = pu | jnp.int32(b)
        thr = cand ^ jnp.int32(_INT_MIN)
        c = jnp.sum((key >= thr).astype(jnp.float32), axis=1, keepdims=True)
        pu = jnp.where(c >= kf, cand, pu)
    tau_key = pu ^ jnp.int32(_INT_MIN)

    sel = key > tau_key
    cnt_gt = jnp.sum(sel.astype(jnp.float32), axis=1)
    sum_gt = jnp.sum(jnp.where(sel, _softplus(x), 0.0), axis=1)

    tau_bits = jnp.where(tau_key >= 0, tau_key,
                         (~tau_key) | jnp.int32(_INT_MIN))
    tau_f = lax.bitcast_convert_type(tau_bits, jnp.float32)[:, 0]

    l_neg = (sum_gt + (kf - cnt_gt) * _softplus(tau_f)) / kf

    for cp in copies:
        cp.wait()
    off = (t & jnp.int32(127))[:, :, None]
    rmod = lax.broadcasted_iota(jnp.int32, (rows, 1, 1), 0) & jnp.int32(7)
    sub = lax.broadcasted_iota(jnp.int32, (rows, 8, _WIN), 1)
    lane = lax.broadcasted_iota(jnp.int32, (rows, 8, _WIN), 2)
    hit = (sub == rmod) & (lane == off)
    pos_cnt = jnp.sum(jnp.sum(jnp.where(hit, win_ref[...], 0.0), axis=2),
                      axis=1)

    l = _DELTA * pos_cnt * _softplus(-pos_logit) + l_neg
    out_ref[...] = jnp.broadcast_to(l[:, None], (rows, 128))


def kernel(logits, targets, co_cnts):
    b, n = logits.shape
    k = int(_R_FRAC * (n - 1))
    rpb = 32
    tgt2 = targets.reshape(b, 1)
    out = pl.pallas_call(
        functools.partial(_body, k=k, n=n),
        grid=(b // rpb,),
        in_specs=[
            pl.BlockSpec((rpb, 1), lambda i: (i, 0),
                         memory_space=pltpu.MemorySpace.SMEM),
            pl.BlockSpec((rpb, 1), lambda i: (i, 0)),
            pl.BlockSpec((rpb, n), lambda i: (i, 0)),
            pl.BlockSpec(memory_space=pltpu.MemorySpace.HBM),
        ],
        out_specs=pl.BlockSpec((rpb, 128), lambda i: (i, 0)),
        out_shape=jax.ShapeDtypeStruct((b, 128), jnp.float32),
        scratch_shapes=[
            pltpu.VMEM((rpb, 8, _WIN), jnp.float32),
            pltpu.SemaphoreType.DMA,
        ],
    )(tgt2, tgt2, logits, co_cnts)
    return jnp.mean(out[:, 0])
